# baseline (speedup 1.0000x reference)
import functools

import jax
import jax.numpy as jnp
from jax import lax
from jax.experimental import pallas as pl
from jax.experimental.pallas import tpu as pltpu
from jax.experimental.pallas import tpu_sc as plsc

NUM_CORES = 2
NUM_SUBCORES = 16
NUM_WORKERS = NUM_CORES * NUM_SUBCORES
LANES = 16

BLK_B = 128
TPITCH = BLK_B + 1


def _build_gather(batch: int, hist: int, embed_dim: int):
    assert batch % (NUM_WORKERS * BLK_B) == 0
    batch_per_w = batch // NUM_WORKERS
    b_per_w = batch_per_w * hist
    nbloc = batch_per_w // BLK_B
    n_blocks = nbloc * hist
    assert n_blocks % 2 == 0
    n_outer = n_blocks // 2
    eg = embed_dim // LANES
    mesh = plsc.VectorSubcoreMesh(
        core_axis_name="c", subcore_axis_name="s",
        num_cores=NUM_CORES, num_subcores=NUM_SUBCORES)

    @functools.partial(
        pl.kernel,
        out_type=jax.ShapeDtypeStruct(
            (hist, embed_dim // 8, batch // 128, 8, 128), jnp.float32),
        mesh=mesh,
        scratch_types=[
            pltpu.VMEM((b_per_w,), jnp.int32),
            pltpu.VMEM((BLK_B,), jnp.int32),
            pltpu.VMEM((BLK_B,), jnp.int32),
            pltpu.VMEM((BLK_B, embed_dim), jnp.float32),
            pltpu.VMEM((BLK_B, embed_dim), jnp.float32),
            pltpu.VMEM((embed_dim // 8, 8, TPITCH), jnp.float32),
            pltpu.VMEM((embed_dim // 8, 8, TPITCH), jnp.float32),
            pltpu.SemaphoreType.DMA,
            pltpu.SemaphoreType.DMA,
            pltpu.SemaphoreType.DMA,
            pltpu.SemaphoreType.DMA,
        ],
        compiler_params=pltpu.CompilerParams(
            use_tc_tiling_on_sc=False, needs_layout_passes=False),
    )
    def gather_kernel(x_hbm, table_hbm, out_hbm, idx_v, cb0, cb1,
                      gb0, gb1, tb0, tb1, g_sem0, g_sem1, o_sem0, o_sem1):
        wid = lax.axis_index("s") * NUM_CORES + lax.axis_index("c")
        base = wid * b_per_w
        batch_base = wid * batch_per_w
        cbufs = (cb0, cb1)
        gbufs = (gb0, gb1)
        tbufs = (tb0, tb1)
        g_sems = (g_sem0, g_sem1)
        o_sems = (o_sem0, o_sem1)

        pltpu.sync_copy(x_hbm.at[pl.ds(base, b_per_w)], idx_v)

        lane = lax.iota(jnp.int32, LANES)
        lane_h = lane * hist

        def fire(blk, par):
            h = blk // nbloc
            bloc = blk % nbloc
            cbuf = cbufs[par]
            col0 = bloc * (BLK_B * hist) + h
            for j in range(BLK_B // LANES):
                vec = plsc.load_gather(idx_v, [lane_h + (col0 + j * LANES * hist)])
                cbuf[pl.ds(j * LANES, LANES)] = vec
            pltpu.async_copy(table_hbm.at[cbufs[par]], gbufs[par], g_sems[par])

        def drain_gather(par):
            pltpu.make_async_copy(
                table_hbm.at[pl.ds(0, BLK_B)], gbufs[par], g_sems[par]).wait()

        et_vecs = [(lane + q * LANES) // 8 for q in range(eg)]
        e8_vecs = [(lane + q * LANES) % 8 for q in range(eg)]

        def transpose(par):
            gbuf = gbufs[par]
            tbuf = tbufs[par]

            @plsc.parallel_loop(0, BLK_B, step=1, unroll=16)
            def _(b):
                bb = lax.broadcast(b, (LANES,))
                for q in range(eg):
                    seg = gbuf[b, pl.ds(q * LANES, LANES)]
                    plsc.store_scatter(tbuf, [et_vecs[q], e8_vecs[q], bb], seg)

        def store(blk, par):
            h = blk // nbloc
            bloc = blk % nbloc
            bt = wid * nbloc + bloc
            pltpu.async_copy(
                tbufs[par].at[:, :, pl.ds(0, BLK_B)],
                out_hbm.at[h, :, bt], o_sems[par])

        def wait_store(par):
            pltpu.make_async_copy(
                tbufs[par].at[:, :, pl.ds(0, BLK_B)],
                out_hbm.at[0, :, 0], o_sems[par]).wait()

        fire(0, 0)

        def outer(g, carry):
            for par in (0, 1):
                i = 2 * g + par
                if par == 0:
                    fire(i + 1, 1)
                else:
                    @pl.when(g < n_outer - 1)
                    def _():
                        fire(i + 1, 0)
                drain_gather(par)
                @pl.when(g > 0)
                def _():
                    wait_store(par)
                transpose(par)
                store(i, par)
            return carry

        lax.fori_loop(0, n_outer, outer, 0)
        wait_store(0)
        wait_store(1)

    return gather_kernel


@jax.jit
def kernel(x, table):
    batch, hist = x.shape
    _, embed_dim = table.shape
    flat_idx = x.reshape(-1).astype(jnp.int32)
    out5 = _build_gather(batch, hist, embed_dim)(flat_idx, table)
    out_t = jnp.transpose(out5, (2, 4, 0, 1, 3))
    return out_t.reshape(batch, hist, embed_dim)


# speedup vs baseline: 1.0109x; 1.0109x over previous
import functools

import jax
import jax.numpy as jnp
from jax import lax
from jax.experimental import pallas as pl
from jax.experimental.pallas import tpu as pltpu
from jax.experimental.pallas import tpu_sc as plsc

NUM_CORES = 2
NUM_SUBCORES = 16
NUM_WORKERS = NUM_CORES * NUM_SUBCORES
LANES = 16

BLK_B = 128
TPITCH = BLK_B + 1


def _build_gather(batch: int, hist: int, embed_dim: int):
    assert batch % (NUM_WORKERS * BLK_B) == 0
    batch_per_w = batch // NUM_WORKERS
    b_per_w = batch_per_w * hist
    nbloc = batch_per_w // BLK_B
    n_blocks = nbloc * hist
    assert n_blocks % 2 == 0
    n_outer = n_blocks // 2
    eg = embed_dim // LANES
    mesh = plsc.VectorSubcoreMesh(
        core_axis_name="c", subcore_axis_name="s",
        num_cores=NUM_CORES, num_subcores=NUM_SUBCORES)

    @functools.partial(
        pl.kernel,
        out_type=jax.ShapeDtypeStruct(
            (hist, embed_dim // 8, batch // 128, 8, 128), jnp.float32),
        mesh=mesh,
        scratch_types=[
            pltpu.VMEM((b_per_w,), jnp.int32),
            pltpu.VMEM((BLK_B,), jnp.int32),
            pltpu.VMEM((BLK_B,), jnp.int32),
            pltpu.VMEM((BLK_B, embed_dim), jnp.float32),
            pltpu.VMEM((BLK_B, embed_dim), jnp.float32),
            pltpu.VMEM((embed_dim // 8, 8, TPITCH), jnp.float32),
            pltpu.VMEM((embed_dim // 8, 8, TPITCH), jnp.float32),
            pltpu.SemaphoreType.DMA,
            pltpu.SemaphoreType.DMA,
            pltpu.SemaphoreType.DMA,
            pltpu.SemaphoreType.DMA,
        ],
        compiler_params=pltpu.CompilerParams(
            use_tc_tiling_on_sc=False, needs_layout_passes=False),
    )
    def gather_kernel(x_hbm, table_hbm, out_hbm, idx_v, cb0, cb1,
                      gb0, gb1, tb0, tb1, g_sem0, g_sem1, o_sem0, o_sem1):
        wid = lax.axis_index("s") * NUM_CORES + lax.axis_index("c")
        base = wid * b_per_w
        batch_base = wid * batch_per_w
        cbufs = (cb0, cb1)
        gbufs = (gb0, gb1)
        tbufs = (tb0, tb1)
        g_sems = (g_sem0, g_sem1)
        o_sems = (o_sem0, o_sem1)

        pltpu.sync_copy(x_hbm.at[pl.ds(base, b_per_w)], idx_v)

        lane = lax.iota(jnp.int32, LANES)
        lane_h = lane * hist

        def fire(blk, par):
            h = blk // nbloc
            bloc = blk % nbloc
            cbuf = cbufs[par]
            col0 = bloc * (BLK_B * hist) + h
            for j in range(BLK_B // LANES):
                vec = plsc.load_gather(idx_v, [lane_h + (col0 + j * LANES * hist)])
                cbuf[pl.ds(j * LANES, LANES)] = vec
            pltpu.async_copy(table_hbm.at[cbufs[par]], gbufs[par], g_sems[par])

        def drain_gather(par):
            pltpu.make_async_copy(
                table_hbm.at[pl.ds(0, BLK_B)], gbufs[par], g_sems[par]).wait()

        et_vecs = [(lane + q * LANES) // 8 for q in range(eg)]
        e8_vecs = [(lane + q * LANES) % 8 for q in range(eg)]

        def transpose(par):
            gbuf = gbufs[par]
            tbuf = tbufs[par]

            @plsc.parallel_loop(0, BLK_B, step=1, unroll=8)
            def _(b):
                bb = lax.broadcast(b, (LANES,))
                for q in range(eg):
                    seg = gbuf[b, pl.ds(q * LANES, LANES)]
                    plsc.store_scatter(tbuf, [et_vecs[q], e8_vecs[q], bb], seg)

        def store(blk, par):
            h = blk // nbloc
            bloc = blk % nbloc
            bt = wid * nbloc + bloc
            pltpu.async_copy(
                tbufs[par].at[:, :, pl.ds(0, BLK_B)],
                out_hbm.at[h, :, bt], o_sems[par])

        def wait_store(par):
            pltpu.make_async_copy(
                tbufs[par].at[:, :, pl.ds(0, BLK_B)],
                out_hbm.at[0, :, 0], o_sems[par]).wait()

        fire(0, 0)

        def outer(g, carry):
            for par in (0, 1):
                i = 2 * g + par
                if par == 0:
                    fire(i + 1, 1)
                else:
                    @pl.when(g < n_outer - 1)
                    def _():
                        fire(i + 1, 0)
                drain_gather(par)
                @pl.when(g > 0)
                def _():
                    wait_store(par)
                transpose(par)
                store(i, par)
            return carry

        lax.fori_loop(0, n_outer, outer, 0)
        wait_store(0)
        wait_store(1)

    return gather_kernel


@jax.jit
def kernel(x, table):
    batch, hist = x.shape
    _, embed_dim = table.shape
    flat_idx = x.reshape(-1).astype(jnp.int32)
    out5 = _build_gather(batch, hist, embed_dim)(flat_idx, table)
    out_t = jnp.transpose(out5, (2, 4, 0, 1, 3))
    return out_t.reshape(batch, hist, embed_dim)
